# baseline (speedup 1.0000x reference)
import functools

import jax
import jax.numpy as jnp
import numpy as np
from jax.experimental import pallas as pl
from jax.experimental.pallas import tpu as pltpu

HIDDEN = 2048
H = 16
HKV = 4
G = H // HKV
D = 64
BS = 64
SBLK = 16
WIN = 512
THETA = 10000.0
NEG = -1e9

RB = 256
TQ = 256


def _rope2d(x, cosb, sinb):
    j = jax.lax.broadcasted_iota(jnp.int32, x.shape, 1) % 64
    lo = jnp.roll(x, -32, axis=1)
    hi = jnp.roll(x, 32, axis=1)
    partner = jnp.where(j < 32, lo, hi)
    return x * cosb + partner * sinb


def _k1_body(hs_ref, wq_ref, wk_ref, wv_ref, wg_ref, cq_ref, sq_ref,
             ck_ref, sk_ref, q_ref, k_ref, v_ref, g_ref, kc_ref, vc_ref):
    nt = (((1,), (1,)), ((), ()))
    hb = hs_ref[:]
    q = jax.lax.dot_general(hb, wq_ref[:], nt,
                            preferred_element_type=jnp.float32)
    q_ref[:] = _rope2d(q, cq_ref[:], sq_ref[:]) * (D ** -0.5)
    k = jax.lax.dot_general(hb, wk_ref[:], nt,
                            preferred_element_type=jnp.float32)
    kr = _rope2d(k, ck_ref[:], sk_ref[:])
    k_ref[:] = kr.astype(jnp.bfloat16)
    v = jax.lax.dot_general(hb, wv_ref[:], nt,
                            preferred_element_type=jnp.float32)
    v_ref[:] = v.astype(jnp.bfloat16)
    g_ref[:] = jax.nn.sigmoid(
        jax.lax.dot_general(hb, wg_ref[:], nt,
                            preferred_element_type=jnp.float32))
    nc = RB // BS
    ci = jax.lax.broadcasted_iota(jnp.int32, (nc, RB), 0)
    ri = jax.lax.broadcasted_iota(jnp.int32, (nc, RB), 1)
    P = jnp.where(ri // BS == ci, 1.0 / BS, 0.0).astype(jnp.float32)
    kc_ref[0] = jnp.dot(P, kr, preferred_element_type=jnp.float32)
    vc_ref[0] = jnp.dot(P, v, preferred_element_type=jnp.float32)


def _softmax_rows(s):
    m = jnp.max(s, axis=-1, keepdims=True)
    e = jnp.exp(s - m)
    return e / jnp.sum(e, axis=-1, keepdims=True)


def _attend_one(qi, r0, kw, sw, swpad, nc, sblk, q_ref, k_ref, v_ref,
                kc_ref, vc_ref, g_ref, wo_ref, e9_ref, swab_ref, crel_ref):
    trow = qi * TQ + jax.lax.broadcasted_iota(jnp.int32, (TQ, 1), 0)
    c32 = jax.lax.broadcasted_iota(jnp.int32, (TQ, nc), 1)
    vis = trow >= (c32 + 1) * BS - 1
    selectable = c32 * BS <= trow
    cur = c32 == trow // BS

    nt = (((1,), (1,)), ((), ()))
    ohs = []
    for h in range(HKV):
        kch = kc_ref[:, h * D:(h + 1) * D]
        vch = vc_ref[:, h * D:(h + 1) * D]
        kh = k_ref[:kw, h * D:(h + 1) * D]
        vh = v_ref[:kw, h * D:(h + 1) * D]
        imp = jnp.zeros((TQ, nc), jnp.float32)
        o_cmp = []
        for g in range(G):
            hd = h * G + g
            qt = q_ref[r0:r0 + TQ, hd * D:(hd + 1) * D]
            sc = jax.lax.dot_general(qt, kch, nt,
                                     preferred_element_type=jnp.float32)
            p = _softmax_rows(jnp.where(vis, sc, NEG))
            p = jnp.where(vis, p, 0.0)
            imp = imp + p
            o_cmp.append(jnp.dot(p, vch,
                                 preferred_element_type=jnp.float32))
        impv = jnp.where(selectable, imp + jnp.where(cur, 1e9, 0.0), NEG)
        a = impv[:, None, :]
        b = impv[:, :, None]
        cpi = jax.lax.broadcasted_iota(jnp.int32, (1, nc, nc), 2)
        ci = jax.lax.broadcasted_iota(jnp.int32, (1, nc, nc), 1)
        gt = (a > b).astype(jnp.float32)
        eq = ((a == b) & (cpi < ci)).astype(jnp.float32)
        rank = jnp.sum(gt + eq, axis=2)
        sel = (rank < sblk).astype(jnp.float32)
        selbias = jnp.dot(sel - 1.0, e9_ref[:, :kw],
                          preferred_element_type=jnp.float32)
        bias_d = selbias[:, kw - TQ:] + crel_ref[:]

        for g in range(G):
            hd = h * G + g
            qt = q_ref[r0:r0 + TQ,
                       hd * D:(hd + 1) * D].astype(jnp.bfloat16)
            s = jax.lax.dot_general(qt, kh, nt,
                                    preferred_element_type=jnp.float32)
            sd = s[:, kw - TQ:] + bias_d
            md = jnp.max(sd, axis=-1, keepdims=True)
            if kw > TQ:
                sm = s[:, :kw - TQ] + selbias[:, :kw - TQ]
                m = jnp.maximum(jnp.max(sm, axis=-1, keepdims=True), md)
                emf = jnp.exp(sm - m)
                edf = jnp.exp(sd - m)
                d = (jnp.sum(emf, axis=-1, keepdims=True)
                     + jnp.sum(edf, axis=-1, keepdims=True))
                o_slc = (jnp.dot(emf.astype(jnp.bfloat16), vh[:kw - TQ],
                                 preferred_element_type=jnp.float32)
                         + jnp.dot(edf.astype(jnp.bfloat16), vh[kw - TQ:],
                                   preferred_element_type=jnp.float32)) / d
            else:
                edf = jnp.exp(sd - md)
                d = jnp.sum(edf, axis=-1, keepdims=True)
                o_slc = jnp.dot(edf.astype(jnp.bfloat16), vh,
                                preferred_element_type=jnp.float32) / d
            sw_s = s[:, kw - sw:] + swab_ref[:, swpad - sw:]
            mw = jnp.max(sw_s, axis=-1, keepdims=True)
            ewf = jnp.exp(sw_s - mw)
            dw = jnp.sum(ewf, axis=-1, keepdims=True)
            o_swa = jnp.dot(ewf.astype(jnp.bfloat16), vh[kw - sw:],
                            preferred_element_type=jnp.float32) / dw
            gb = g_ref[r0:r0 + TQ, 3 * hd:3 * hd + 3]
            oh = (gb[:, 0:1] * o_cmp[g] + gb[:, 1:2] * o_slc
                  + gb[:, 2:3] * o_swa)
            ohs.append(oh.astype(jnp.bfloat16))
    return jax.lax.dot_general(jnp.concatenate(ohs, axis=1), wo_ref[:],
                               nt, preferred_element_type=jnp.float32)


def _k2_multi(qlo, nqi, nc, sblk, swpad, q_ref, k_ref, v_ref, kc_ref,
              vc_ref, g_ref, wo_ref, e9_ref, swab_ref, crel_ref, out_ref):
    for ql in range(nqi):
        qi = qlo + ql
        kw = (qi + 1) * TQ
        sw = min(WIN + TQ, kw)
        out_ref[ql * TQ:(ql + 1) * TQ, :] = _attend_one(
            qi, ql * TQ, kw, sw, swpad, nc, sblk, q_ref, k_ref, v_ref,
            kc_ref, vc_ref, g_ref, wo_ref, e9_ref, swab_ref[ql], crel_ref)


def kernel(hidden_states, Wq, Wk, Wv, Wg, Wo):
    B, T, HID = hidden_states.shape
    hs = hidden_states.reshape(T, HID)
    nc = T // BS
    sblk = min(SBLK, nc)
    nrb = T // RB
    nqb = T // TQ

    inv = 1.0 / (THETA ** (np.arange(32, dtype=np.float32) / 32.0))
    fr = np.outer(np.arange(T, dtype=np.float32), inv)
    cosT = np.cos(fr).astype(np.float32)
    sinT = np.sin(fr).astype(np.float32)
    cq = np.tile(np.concatenate([cosT, cosT], axis=1), (1, H))
    sq = np.tile(np.concatenate([-sinT, sinT], axis=1), (1, H))
    ck = np.tile(np.concatenate([cosT, cosT], axis=1), (1, HKV))
    sk = np.tile(np.concatenate([-sinT, sinT], axis=1), (1, HKV))

    full = lambda shape: pl.BlockSpec(shape, lambda i: tuple(0 for _ in shape))
    q2d, k2d, v2d, g2d, kc3, vc3 = pl.pallas_call(
        _k1_body,
        grid=(nrb,),
        in_specs=[
            pl.BlockSpec((RB, HID), lambda i: (i, 0)),
            full((H * D, HID)), full((HKV * D, HID)), full((HKV * D, HID)),
            full((H * 3, HID)),
            pl.BlockSpec((RB, H * D), lambda i: (i, 0)),
            pl.BlockSpec((RB, H * D), lambda i: (i, 0)),
            pl.BlockSpec((RB, HKV * D), lambda i: (i, 0)),
            pl.BlockSpec((RB, HKV * D), lambda i: (i, 0)),
        ],
        out_specs=[
            pl.BlockSpec((RB, H * D), lambda i: (i, 0)),
            pl.BlockSpec((RB, HKV * D), lambda i: (i, 0)),
            pl.BlockSpec((RB, HKV * D), lambda i: (i, 0)),
            pl.BlockSpec((RB, H * 3), lambda i: (i, 0)),
            pl.BlockSpec((1, RB // BS, HKV * D), lambda i: (i, 0, 0)),
            pl.BlockSpec((1, RB // BS, HKV * D), lambda i: (i, 0, 0)),
        ],
        out_shape=[
            jax.ShapeDtypeStruct((T, H * D), jnp.float32),
            jax.ShapeDtypeStruct((T, HKV * D), jnp.bfloat16),
            jax.ShapeDtypeStruct((T, HKV * D), jnp.bfloat16),
            jax.ShapeDtypeStruct((T, H * 3), jnp.float32),
            jax.ShapeDtypeStruct((nrb, RB // BS, HKV * D), jnp.float32),
            jax.ShapeDtypeStruct((nrb, RB // BS, HKV * D), jnp.float32),
        ],
    )(hs, Wq, Wk, Wv, Wg, cq, sq, ck, sk)

    kc2 = kc3.reshape(nc, HKV * D)
    vc2 = vc3.reshape(nc, HKV * D)
    wob = Wo.astype(jnp.bfloat16)
    eci = np.arange(nc)[:, None]
    eti = np.arange(T)[None, :]
    e9 = np.where(eti // BS == eci, 1e9, 0.0).astype(np.float32)

    rr = np.arange(TQ)[:, None]
    crel = np.where(rr >= np.arange(TQ)[None, :], 0.0,
                    NEG).astype(np.float32)

    outs = []
    for qlo in range(0, nqb, 4):
        nqi = min(4, nqb - qlo)
        kwmax = (qlo + nqi) * TQ
        swpad = min(WIN + TQ, kwmax)
        swab = np.full((nqi, TQ, swpad), NEG, np.float32)
        for ql in range(nqi):
            qi = qlo + ql
            kw = (qi + 1) * TQ
            sw = min(WIN + TQ, kw)
            tt = qi * TQ + rr
            ccw = (kw - sw) + np.arange(sw)[None, :]
            swab[ql, :, swpad - sw:] = np.where(
                (tt >= ccw) & (tt - ccw < WIN), 0.0, NEG)
        outs.append(pl.pallas_call(
            functools.partial(_k2_multi, qlo, nqi, nc, sblk, swpad),
            grid=(1,),
            in_specs=[
                pl.BlockSpec((nqi * TQ, H * D),
                             lambda i, _b=qlo // 4: (_b, 0)),
                pl.BlockSpec((kwmax, HKV * D), lambda i: (0, 0)),
                pl.BlockSpec((kwmax, HKV * D), lambda i: (0, 0)),
                pl.BlockSpec((nc, HKV * D), lambda i: (0, 0)),
                pl.BlockSpec((nc, HKV * D), lambda i: (0, 0)),
                pl.BlockSpec((nqi * TQ, H * 3),
                             lambda i, _b=qlo // 4: (_b, 0)),
                pl.BlockSpec((HID, H * D), lambda i: (0, 0)),
                pl.BlockSpec((nc, kwmax), lambda i: (0, 0)),
                pl.BlockSpec((nqi, TQ, swpad), lambda i: (0, 0, 0)),
                pl.BlockSpec((TQ, TQ), lambda i: (0, 0)),
            ],
            out_specs=pl.BlockSpec((nqi * TQ, HID), lambda i: (0, 0)),
            out_shape=jax.ShapeDtypeStruct((nqi * TQ, HID), jnp.float32),
        )(q2d, k2d, v2d, kc2, vc2, g2d, wob, e9[:, :kwmax], swab, crel))

    out = jnp.concatenate(outs, axis=0) if len(outs) > 1 else outs[0]
    return out.reshape(B, T, HID)
